# speedup vs baseline: 5.5887x; 5.5887x over previous
import functools

import jax
import jax.numpy as jnp
from jax import lax
from jax.experimental import pallas as pl
from jax.experimental.pallas import tpu as pltpu
from jax.experimental.pallas import tpu_sc as plsc

_NC = 2
_NS = 16
_CH = 128
_BN = 1000
_G = 128


@functools.lru_cache(maxsize=None)
def _make_sc_seg_sum(N, E, H):
    n_chunks = E // _CH
    assert n_chunks * _CH == E
    per_core = n_chunks // _NC
    k_full = per_core // _NS
    rem = per_core % _NS
    rps = (N // _NS) // 16 * 16
    tail = N - rps * _NS
    assert tail % 16 == 0

    mesh = plsc.VectorSubcoreMesh(
        core_axis_name="c", subcore_axis_name="s",
        num_cores=_NC, num_subcores=_NS)

    @functools.partial(
        pl.kernel,
        mesh=mesh,
        out_type=jax.ShapeDtypeStruct((_NC, N, H), jnp.float32),
        scratch_types=[
            pltpu.VMEM((_CH,), jnp.int32),
            pltpu.VMEM((_CH,), jnp.int32),
            pltpu.VMEM((_CH, H), jnp.float32),
            pltpu.VMEM((16, H), jnp.float32),
            pltpu.VMEM_SHARED((N, H), jnp.float32),
            pltpu.SemaphoreType.DMA,
        ],
    )
    def seg(h_hbm, src_hbm, dst_hbm, out_hbm, srcv, dstv, rowsv, zbuf, acc, sem):
        c = lax.axis_index("c")
        s = lax.axis_index("s")

        for r in range(16):
            for q in range(H // 16):
                zbuf[r, pl.ds(q * 16, 16)] = jnp.zeros((16,), jnp.float32)

        zbase = s * rps

        def zero_body(j, carry):
            pltpu.sync_copy(zbuf, acc.at[pl.ds(zbase + j * 16, 16)])
            return carry

        lax.fori_loop(0, rps // 16, zero_body, 0)
        if tail:
            @pl.when(s == _NS - 1)
            def _():
                for j in range(tail // 16):
                    pltpu.sync_copy(zbuf, acc.at[pl.ds(N - tail + j * 16, 16)])

        plsc.subcore_barrier()

        def edge_chunk(cid):
            base = cid * _CH
            pltpu.sync_copy(src_hbm.at[pl.ds(base, _CH)], srcv)
            pltpu.sync_copy(dst_hbm.at[pl.ds(base, _CH)], dstv)
            pltpu.async_copy(h_hbm.at[srcv], rowsv, sem).wait()
            pltpu.sync_copy(rowsv, acc.at[dstv], add=True)

        cbase = c * per_core + s

        def edge_body(k, carry):
            edge_chunk(cbase + _NS * k)
            return carry

        lax.fori_loop(0, k_full, edge_body, 0)
        if rem:
            @pl.when(s < rem)
            def _():
                edge_chunk(cbase + _NS * k_full)

        plsc.subcore_barrier()

        pltpu.sync_copy(acc.at[pl.ds(zbase, rps)],
                        out_hbm.at[c, pl.ds(zbase, rps)])
        if tail:
            @pl.when(s == _NS - 1)
            def _():
                pltpu.sync_copy(acc.at[pl.ds(N - tail, tail)],
                                out_hbm.at[c, pl.ds(N - tail, tail)])

    return seg


def _rowvec(v):
    return v.reshape(1, v.shape[0])


@functools.lru_cache(maxsize=None)
def _make_encoder(N, D, H):
    def body(x_ref, w_ref, b_ref, o_ref):
        o_ref[...] = (jnp.dot(x_ref[...], w_ref[...],
                              preferred_element_type=jnp.float32)
                      + b_ref[...])

    return pl.pallas_call(
        body,
        grid=(N // _BN,),
        in_specs=[
            pl.BlockSpec((_BN, D), lambda i: (i, 0)),
            pl.BlockSpec((D, H), lambda i: (0, 0)),
            pl.BlockSpec((1, H), lambda i: (0, 0)),
        ],
        out_specs=pl.BlockSpec((_BN, H), lambda i: (i, 0)),
        out_shape=jax.ShapeDtypeStruct((N, H), jnp.float32),
    )


@functools.lru_cache(maxsize=None)
def _make_gin_mlp(N, H):
    def body(p_ref, h_ref, w1, b1, g1, e1, w2, b2, o_ref):
        t = p_ref[0] + p_ref[1] + h_ref[...]
        t = jnp.dot(t, w1[...], preferred_element_type=jnp.float32) + b1[...]
        t = t * g1[...] + e1[...]
        t = jnp.maximum(t, 0.0)
        t = jnp.dot(t, w2[...], preferred_element_type=jnp.float32) + b2[...]
        o_ref[...] = jnp.maximum(t, 0.0)

    wspec = pl.BlockSpec((H, H), lambda i: (0, 0))
    bspec = pl.BlockSpec((1, H), lambda i: (0, 0))
    return pl.pallas_call(
        body,
        grid=(N // _BN,),
        in_specs=[
            pl.BlockSpec((_NC, _BN, H), lambda i: (0, i, 0)),
            pl.BlockSpec((_BN, H), lambda i: (i, 0)),
            wspec, bspec, bspec, bspec, wspec, bspec,
        ],
        out_specs=pl.BlockSpec((_BN, H), lambda i: (i, 0)),
        out_shape=jax.ShapeDtypeStruct((N, H), jnp.float32),
    )


@functools.lru_cache(maxsize=None)
def _make_pool_head(N, H):
    nsteps = N // _BN

    def body(h_ref, b_ref, wf1, bf1, gf1, ef1, wf2, bf2, gf2, ef2, wo, bo,
             o_ref):
        i = pl.program_id(0)

        @pl.when(i == 0)
        def _():
            o_ref[...] = jnp.zeros_like(o_ref)

        onehot = (b_ref[...] == lax.broadcasted_iota(
            jnp.int32, (_BN, _G), 1)).astype(jnp.float32)
        o_ref[...] += lax.dot_general(
            onehot, h_ref[...], (((0,), (0,)), ((), ())),
            preferred_element_type=jnp.float32)

        @pl.when(i == nsteps - 1)
        def _():
            z = o_ref[...]
            z = jnp.dot(z, wf1[...], preferred_element_type=jnp.float32) + bf1[...]
            z = z * gf1[...] + ef1[...]
            z = jnp.maximum(z, 0.0)
            z = jnp.dot(z, wf2[...], preferred_element_type=jnp.float32) + bf2[...]
            z = z * gf2[...] + ef2[...]
            o_ref[...] = (jnp.dot(z, wo[...],
                                  preferred_element_type=jnp.float32)
                          + bo[...])

    wspec = pl.BlockSpec((H, H), lambda i: (0, 0))
    bspec = pl.BlockSpec((1, H), lambda i: (0, 0))
    return pl.pallas_call(
        body,
        grid=(nsteps,),
        in_specs=[
            pl.BlockSpec((_BN, H), lambda i: (i, 0)),
            pl.BlockSpec((_BN, 1), lambda i: (i, 0)),
            wspec, bspec, bspec, bspec, wspec, bspec, bspec, bspec,
            wspec, bspec,
        ],
        out_specs=pl.BlockSpec((_G, H), lambda i: (0, 0)),
        out_shape=jax.ShapeDtypeStruct((_G, H), jnp.float32),
    )


def kernel(x, edge_index, batch, W_enc, b_enc, W1, b1, g1, be1, W2, b2,
           Wf1, bf1, gf1, bef1, Wf2, bf2, gf2, bef2, Wout, bout):
    N, D = x.shape
    H = W_enc.shape[1]
    L = W1.shape[0]
    C = Wout.shape[1]
    E = edge_index.shape[1]

    src = edge_index[0]
    dst = edge_index[1]

    h = _make_encoder(N, D, H)(x, W_enc, _rowvec(b_enc))

    seg = _make_sc_seg_sum(N, E, H)
    mlp = _make_gin_mlp(N, H)
    for l in range(L):
        partials = seg(h, src, dst)
        h = mlp(partials, h, W1[l], _rowvec(b1[l]), _rowvec(g1[l]),
                _rowvec(be1[l]), W2[l], _rowvec(b2[l]))

    wout_p = jnp.pad(Wout, ((0, 0), (0, H - C)))
    bout_p = _rowvec(jnp.pad(bout, (0, H - C)))
    z = _make_pool_head(N, H)(
        h, batch.reshape(N, 1), Wf1, _rowvec(bf1), _rowvec(gf1),
        _rowvec(bef1), Wf2, _rowvec(bf2), _rowvec(gf2), _rowvec(bef2),
        wout_p, bout_p)
    return z[:, :C]


# speedup vs baseline: 11.3153x; 2.0247x over previous
import functools

import jax
import jax.numpy as jnp
from jax import lax
from jax.experimental import pallas as pl
from jax.experimental.pallas import tpu as pltpu
from jax.experimental.pallas import tpu_sc as plsc

_NC = 2
_NS = 16
_CH = 128
_BN = 1000
_G = 128


@functools.lru_cache(maxsize=None)
def _make_sc_seg_sum(N, E, H):
    n_chunks = E // _CH
    assert n_chunks * _CH == E
    ntiles = _NC * _NS
    nt_base = n_chunks // ntiles
    nt_rem = n_chunks - nt_base * ntiles
    nt_max = nt_base + (1 if nt_rem else 0)
    ph = 40
    nphases = -(-nt_max // ph)
    rps = (N // _NS) // 16 * 16
    tail = N - rps * _NS
    assert tail % 16 == 0 and tail <= 16

    mesh = plsc.VectorSubcoreMesh(
        core_axis_name="c", subcore_axis_name="s",
        num_cores=_NC, num_subcores=_NS)

    @functools.partial(
        pl.kernel,
        mesh=mesh,
        out_type=jax.ShapeDtypeStruct((_NC, N, H), jnp.float32),
        scratch_types=[
            pltpu.VMEM((ph, 1, _CH), jnp.int32),
            pltpu.VMEM((ph, 1, _CH), jnp.int32),
            pltpu.VMEM((2, _CH, H), jnp.float32),
            pltpu.VMEM((16, H), jnp.float32),
            pltpu.VMEM_SHARED((N, H), jnp.float32),
            pltpu.SemaphoreType.DMA,
        ],
    )
    def seg(h_hbm, src_hbm, dst_hbm, out_hbm, srcv, dstv, rowsv, zbuf, acc, sem):
        c = lax.axis_index("c")
        s = lax.axis_index("s")
        gid = c * _NS + s
        nt = jnp.where(gid < nt_rem, nt_max, nt_base)
        tbase = jnp.where(gid < nt_rem, gid * nt_max,
                          gid * nt_base + nt_rem)

        for r in range(16):
            for q in range(H // 16):
                zbuf[r, pl.ds(q * 16, 16)] = jnp.zeros((16,), jnp.float32)
        zbase = s * rps

        def zero_body(j, carry):
            pltpu.sync_copy(zbuf, acc.at[pl.ds(zbase + j * 16, 16)])
            return carry

        lax.fori_loop(0, rps // 16, zero_body, 0)
        if tail:
            @pl.when(s == _NS - 1)
            def _():
                pltpu.sync_copy(zbuf.at[pl.ds(0, tail)],
                                acc.at[pl.ds(N - tail, tail)])

        plsc.subcore_barrier()

        def gather(k):
            return pltpu.make_async_copy(
                h_hbm.at[srcv.at[k, 0]], rowsv.at[k % 2], sem)

        for p in range(nphases):
            pbase = tbase + p * ph
            pltpu.sync_copy(src_hbm.at[pl.ds(pbase, ph)], srcv)
            pltpu.sync_copy(dst_hbm.at[pl.ds(pbase, ph)], dstv)
            cnt = jnp.minimum(nt - p * ph, ph)

            gather(0).start()

            def edge_body(k, carry, cnt=cnt):
                @pl.when(k + 1 < cnt)
                def _():
                    gather(k + 1).start()
                gather(k).wait()
                pltpu.sync_copy(rowsv.at[k % 2], acc.at[dstv.at[k, 0]],
                                add=True)
                return carry

            lax.fori_loop(0, cnt, edge_body, 0)

        plsc.subcore_barrier()

        pltpu.sync_copy(acc.at[pl.ds(zbase, rps)],
                        out_hbm.at[c, pl.ds(zbase, rps)])
        if tail:
            @pl.when(s == _NS - 1)
            def _():
                pltpu.sync_copy(acc.at[pl.ds(N - tail, tail)],
                                out_hbm.at[c, pl.ds(N - tail, tail)])

    return seg


def _rowvec(v):
    return v.reshape(1, v.shape[0])


@functools.lru_cache(maxsize=None)
def _make_encoder(N, D, H):
    def body(x_ref, w_ref, b_ref, o_ref):
        o_ref[...] = (jnp.dot(x_ref[...], w_ref[...],
                              preferred_element_type=jnp.float32)
                      + b_ref[...])

    return pl.pallas_call(
        body,
        grid=(N // _BN,),
        in_specs=[
            pl.BlockSpec((_BN, D), lambda i: (i, 0)),
            pl.BlockSpec((D, H), lambda i: (0, 0)),
            pl.BlockSpec((1, H), lambda i: (0, 0)),
        ],
        out_specs=pl.BlockSpec((_BN, H), lambda i: (i, 0)),
        out_shape=jax.ShapeDtypeStruct((N, H), jnp.float32),
    )


@functools.lru_cache(maxsize=None)
def _make_gin_mlp(N, H):
    def body(p_ref, h_ref, w1, b1, g1, e1, w2, b2, o_ref):
        t = p_ref[0] + p_ref[1] + h_ref[...]
        t = jnp.dot(t, w1[...], preferred_element_type=jnp.float32) + b1[...]
        t = t * g1[...] + e1[...]
        t = jnp.maximum(t, 0.0)
        t = jnp.dot(t, w2[...], preferred_element_type=jnp.float32) + b2[...]
        o_ref[...] = jnp.maximum(t, 0.0)

    wspec = pl.BlockSpec((H, H), lambda i: (0, 0))
    bspec = pl.BlockSpec((1, H), lambda i: (0, 0))
    return pl.pallas_call(
        body,
        grid=(N // _BN,),
        in_specs=[
            pl.BlockSpec((_NC, _BN, H), lambda i: (0, i, 0)),
            pl.BlockSpec((_BN, H), lambda i: (i, 0)),
            wspec, bspec, bspec, bspec, wspec, bspec,
        ],
        out_specs=pl.BlockSpec((_BN, H), lambda i: (i, 0)),
        out_shape=jax.ShapeDtypeStruct((N, H), jnp.float32),
    )


@functools.lru_cache(maxsize=None)
def _make_pool_head(N, H):
    nsteps = N // _BN

    def body(h_ref, b_ref, wf1, bf1, gf1, ef1, wf2, bf2, gf2, ef2, wo, bo,
             o_ref):
        i = pl.program_id(0)

        @pl.when(i == 0)
        def _():
            o_ref[...] = jnp.zeros_like(o_ref)

        onehot = (b_ref[...] == lax.broadcasted_iota(
            jnp.int32, (_BN, _G), 1)).astype(jnp.float32)
        o_ref[...] += lax.dot_general(
            onehot, h_ref[...], (((0,), (0,)), ((), ())),
            preferred_element_type=jnp.float32)

        @pl.when(i == nsteps - 1)
        def _():
            z = o_ref[...]
            z = jnp.dot(z, wf1[...], preferred_element_type=jnp.float32) + bf1[...]
            z = z * gf1[...] + ef1[...]
            z = jnp.maximum(z, 0.0)
            z = jnp.dot(z, wf2[...], preferred_element_type=jnp.float32) + bf2[...]
            z = z * gf2[...] + ef2[...]
            o_ref[...] = (jnp.dot(z, wo[...],
                                  preferred_element_type=jnp.float32)
                          + bo[...])

    wspec = pl.BlockSpec((H, H), lambda i: (0, 0))
    bspec = pl.BlockSpec((1, H), lambda i: (0, 0))
    return pl.pallas_call(
        body,
        grid=(nsteps,),
        in_specs=[
            pl.BlockSpec((_BN, H), lambda i: (i, 0)),
            pl.BlockSpec((_BN, 1), lambda i: (i, 0)),
            wspec, bspec, bspec, bspec, wspec, bspec, bspec, bspec,
            wspec, bspec,
        ],
        out_specs=pl.BlockSpec((_G, H), lambda i: (0, 0)),
        out_shape=jax.ShapeDtypeStruct((_G, H), jnp.float32),
    )


def kernel(x, edge_index, batch, W_enc, b_enc, W1, b1, g1, be1, W2, b2,
           Wf1, bf1, gf1, bef1, Wf2, bf2, gf2, bef2, Wout, bout):
    N, D = x.shape
    H = W_enc.shape[1]
    L = W1.shape[0]
    C = Wout.shape[1]
    E = edge_index.shape[1]

    src = jnp.pad(edge_index[0].reshape(E // _CH, 1, _CH), ((0, 48), (0, 0), (0, 0)))
    dst = jnp.pad(edge_index[1].reshape(E // _CH, 1, _CH), ((0, 48), (0, 0), (0, 0)))

    h = _make_encoder(N, D, H)(x, W_enc, _rowvec(b_enc))

    seg = _make_sc_seg_sum(N, E, H)
    mlp = _make_gin_mlp(N, H)
    for l in range(L):
        partials = seg(h, src, dst)
        h = mlp(partials, h, W1[l], _rowvec(b1[l]), _rowvec(g1[l]),
                _rowvec(be1[l]), W2[l], _rowvec(b2[l]))

    wout_p = jnp.pad(Wout, ((0, 0), (0, H - C)))
    bout_p = _rowvec(jnp.pad(bout, (0, H - C)))
    z = _make_pool_head(N, H)(
        h, batch.reshape(N, 1), Wf1, _rowvec(bf1), _rowvec(gf1),
        _rowvec(bef1), Wf2, _rowvec(bf2), _rowvec(gf2), _rowvec(bef2),
        wout_p, bout_p)
    return z[:, :C]


# speedup vs baseline: 11.8951x; 1.0512x over previous
import functools

import jax
import jax.numpy as jnp
from jax import lax
from jax.experimental import pallas as pl
from jax.experimental.pallas import tpu as pltpu
from jax.experimental.pallas import tpu_sc as plsc

_NC = 2
_NS = 16
_CH = 128
_BN = 1000
_G = 128


@functools.lru_cache(maxsize=None)
def _make_sc_seg_sum(N, E, H):
    n_chunks = E // _CH
    assert n_chunks * _CH == E
    ntiles = _NC * _NS
    nt_base = n_chunks // ntiles
    nt_rem = n_chunks - nt_base * ntiles
    nt_max = nt_base + (1 if nt_rem else 0)
    ph = 28
    nphases = -(-nt_max // ph)
    rps = (N // _NS) // 16 * 16
    tail = N - rps * _NS
    assert tail % 16 == 0 and tail <= 16

    mesh = plsc.VectorSubcoreMesh(
        core_axis_name="c", subcore_axis_name="s",
        num_cores=_NC, num_subcores=_NS)

    @functools.partial(
        pl.kernel,
        mesh=mesh,
        out_type=jax.ShapeDtypeStruct((_NC, N, H), jnp.float32),
        scratch_types=[
            pltpu.VMEM((2, ph, 1, _CH), jnp.int32),
            pltpu.VMEM((2, ph, 1, _CH), jnp.int32),
            pltpu.VMEM((2, _CH, H), jnp.float32),
            pltpu.VMEM((16, H), jnp.float32),
            pltpu.VMEM_SHARED((N, H), jnp.float32),
            pltpu.SemaphoreType.DMA,
            pltpu.SemaphoreType.DMA,
            pltpu.SemaphoreType.DMA,
        ],
    )
    def seg(h_hbm, src_hbm, dst_hbm, out_hbm, srcv, dstv, rowsv, zbuf, acc,
            gsem, isem, zsem):
        c = lax.axis_index("c")
        s = lax.axis_index("s")
        gid = c * _NS + s
        nt = jnp.where(gid < nt_rem, nt_max, nt_base)
        tbase = jnp.where(gid < nt_rem, gid * nt_max,
                          gid * nt_base + nt_rem)
        zbase = s * rps

        def stage(p):
            sp = p % 2
            pbase = tbase + p * ph
            return (pltpu.make_async_copy(
                        src_hbm.at[pl.ds(pbase, ph)], srcv.at[sp], isem),
                    pltpu.make_async_copy(
                        dst_hbm.at[pl.ds(pbase, ph)], dstv.at[sp], isem))

        def gather(p, k, buf):
            return pltpu.make_async_copy(
                h_hbm.at[srcv.at[p % 2, k, 0]], rowsv.at[buf], gsem)

        def zero_copy(j):
            return pltpu.make_async_copy(
                zbuf, acc.at[pl.ds(zbase + j * 16, 16)], zsem)

        for d in stage(0):
            d.start()
        for r in range(16):
            for q in range(H // 16):
                zbuf[r, pl.ds(q * 16, 16)] = jnp.zeros((16,), jnp.float32)
        for d in stage(0):
            d.wait()
        gather(0, 0, 0).start()
        if nphases > 1:
            for d in stage(1):
                d.start()

        def zfire(j, carry):
            zero_copy(j).start()
            return carry

        lax.fori_loop(0, rps // 16, zfire, 0)

        def zdrain(j, carry):
            zero_copy(j).wait()
            return carry

        lax.fori_loop(0, rps // 16, zdrain, 0)
        if tail:
            @pl.when(s == _NS - 1)
            def _():
                pltpu.sync_copy(zbuf.at[pl.ds(0, tail)],
                                acc.at[pl.ds(N - tail, tail)])

        plsc.subcore_barrier()

        for p in range(nphases):
            cnt = jnp.minimum(nt - p * ph, ph)

            def edge_body(k, carry, p=p, cnt=cnt):
                @pl.when(k + 1 < cnt)
                def _():
                    gather(p, k + 1, (k + 1) % 2).start()
                gather(p, k, k % 2).wait()
                pltpu.sync_copy(rowsv.at[k % 2], acc.at[dstv.at[p % 2, k, 0]],
                                add=True)
                return carry

            lax.fori_loop(0, cnt, edge_body, 0)

            if p + 1 < nphases:
                for d in stage(p + 1):
                    d.wait()
                gather(p + 1, 0, 0).start()
                if p + 2 < nphases:
                    for d in stage(p + 2):
                        d.start()

        plsc.subcore_barrier()

        pltpu.sync_copy(acc.at[pl.ds(zbase, rps)],
                        out_hbm.at[c, pl.ds(zbase, rps)])
        if tail:
            @pl.when(s == _NS - 1)
            def _():
                pltpu.sync_copy(acc.at[pl.ds(N - tail, tail)],
                                out_hbm.at[c, pl.ds(N - tail, tail)])

    return seg


def _rowvec(v):
    return v.reshape(1, v.shape[0])


@functools.lru_cache(maxsize=None)
def _make_encoder(N, D, H):
    def body(x_ref, w_ref, b_ref, o_ref):
        o_ref[...] = (jnp.dot(x_ref[...], w_ref[...],
                              preferred_element_type=jnp.float32)
                      + b_ref[...])

    return pl.pallas_call(
        body,
        grid=(N // _BN,),
        in_specs=[
            pl.BlockSpec((_BN, D), lambda i: (i, 0)),
            pl.BlockSpec((D, H), lambda i: (0, 0)),
            pl.BlockSpec((1, H), lambda i: (0, 0)),
        ],
        out_specs=pl.BlockSpec((_BN, H), lambda i: (i, 0)),
        out_shape=jax.ShapeDtypeStruct((N, H), jnp.float32),
    )


@functools.lru_cache(maxsize=None)
def _make_gin_mlp(N, H):
    def body(p_ref, h_ref, w1, b1, g1, e1, w2, b2, o_ref):
        t = p_ref[0] + p_ref[1] + h_ref[...]
        t = jnp.dot(t, w1[...], preferred_element_type=jnp.float32) + b1[...]
        t = t * g1[...] + e1[...]
        t = jnp.maximum(t, 0.0)
        t = jnp.dot(t, w2[...], preferred_element_type=jnp.float32) + b2[...]
        o_ref[...] = jnp.maximum(t, 0.0)

    wspec = pl.BlockSpec((H, H), lambda i: (0, 0))
    bspec = pl.BlockSpec((1, H), lambda i: (0, 0))
    return pl.pallas_call(
        body,
        grid=(N // _BN,),
        in_specs=[
            pl.BlockSpec((_NC, _BN, H), lambda i: (0, i, 0)),
            pl.BlockSpec((_BN, H), lambda i: (i, 0)),
            wspec, bspec, bspec, bspec, wspec, bspec,
        ],
        out_specs=pl.BlockSpec((_BN, H), lambda i: (i, 0)),
        out_shape=jax.ShapeDtypeStruct((N, H), jnp.float32),
    )


@functools.lru_cache(maxsize=None)
def _make_gin_mlp_pool_head(N, H):
    nsteps = N // _BN

    def body(p_ref, h_ref, w1, b1, g1, e1, w2, b2, bt_ref,
             wf1, bf1, gf1, ef1, wf2, bf2, gf2, ef2, wo, bo, o_ref):
        i = pl.program_id(0)

        @pl.when(i == 0)
        def _():
            o_ref[...] = jnp.zeros_like(o_ref)

        t = p_ref[0] + p_ref[1] + h_ref[...]
        t = jnp.dot(t, w1[...], preferred_element_type=jnp.float32) + b1[...]
        t = t * g1[...] + e1[...]
        t = jnp.maximum(t, 0.0)
        t = jnp.dot(t, w2[...], preferred_element_type=jnp.float32) + b2[...]
        t = jnp.maximum(t, 0.0)

        onehot = (bt_ref[...] == lax.broadcasted_iota(
            jnp.int32, (_BN, _G), 1)).astype(jnp.float32)
        o_ref[...] += lax.dot_general(
            onehot, t, (((0,), (0,)), ((), ())),
            preferred_element_type=jnp.float32)

        @pl.when(i == nsteps - 1)
        def _():
            z = o_ref[...]
            z = jnp.dot(z, wf1[...], preferred_element_type=jnp.float32) + bf1[...]
            z = z * gf1[...] + ef1[...]
            z = jnp.maximum(z, 0.0)
            z = jnp.dot(z, wf2[...], preferred_element_type=jnp.float32) + bf2[...]
            z = z * gf2[...] + ef2[...]
            o_ref[...] = (jnp.dot(z, wo[...],
                                  preferred_element_type=jnp.float32)
                          + bo[...])

    wspec = pl.BlockSpec((H, H), lambda i: (0, 0))
    bspec = pl.BlockSpec((1, H), lambda i: (0, 0))
    return pl.pallas_call(
        body,
        grid=(nsteps,),
        in_specs=[
            pl.BlockSpec((_NC, _BN, H), lambda i: (0, i, 0)),
            pl.BlockSpec((_BN, H), lambda i: (i, 0)),
            wspec, bspec, bspec, bspec, wspec, bspec,
            pl.BlockSpec((_BN, 1), lambda i: (i, 0)),
            wspec, bspec, bspec, bspec, wspec, bspec, bspec, bspec,
            wspec, bspec,
        ],
        out_specs=pl.BlockSpec((_G, H), lambda i: (0, 0)),
        out_shape=jax.ShapeDtypeStruct((_G, H), jnp.float32),
    )


@functools.lru_cache(maxsize=None)
def _make_pool_head(N, H):
    nsteps = N // _BN

    def body(h_ref, b_ref, wf1, bf1, gf1, ef1, wf2, bf2, gf2, ef2, wo, bo,
             o_ref):
        i = pl.program_id(0)

        @pl.when(i == 0)
        def _():
            o_ref[...] = jnp.zeros_like(o_ref)

        onehot = (b_ref[...] == lax.broadcasted_iota(
            jnp.int32, (_BN, _G), 1)).astype(jnp.float32)
        o_ref[...] += lax.dot_general(
            onehot, h_ref[...], (((0,), (0,)), ((), ())),
            preferred_element_type=jnp.float32)

        @pl.when(i == nsteps - 1)
        def _():
            z = o_ref[...]
            z = jnp.dot(z, wf1[...], preferred_element_type=jnp.float32) + bf1[...]
            z = z * gf1[...] + ef1[...]
            z = jnp.maximum(z, 0.0)
            z = jnp.dot(z, wf2[...], preferred_element_type=jnp.float32) + bf2[...]
            z = z * gf2[...] + ef2[...]
            o_ref[...] = (jnp.dot(z, wo[...],
                                  preferred_element_type=jnp.float32)
                          + bo[...])

    wspec = pl.BlockSpec((H, H), lambda i: (0, 0))
    bspec = pl.BlockSpec((1, H), lambda i: (0, 0))
    return pl.pallas_call(
        body,
        grid=(nsteps,),
        in_specs=[
            pl.BlockSpec((_BN, H), lambda i: (i, 0)),
            pl.BlockSpec((_BN, 1), lambda i: (i, 0)),
            wspec, bspec, bspec, bspec, wspec, bspec, bspec, bspec,
            wspec, bspec,
        ],
        out_specs=pl.BlockSpec((_G, H), lambda i: (0, 0)),
        out_shape=jax.ShapeDtypeStruct((_G, H), jnp.float32),
    )


def kernel(x, edge_index, batch, W_enc, b_enc, W1, b1, g1, be1, W2, b2,
           Wf1, bf1, gf1, bef1, Wf2, bf2, gf2, bef2, Wout, bout):
    N, D = x.shape
    H = W_enc.shape[1]
    L = W1.shape[0]
    C = Wout.shape[1]
    E = edge_index.shape[1]

    src = jnp.pad(edge_index[0].reshape(E // _CH, 1, _CH), ((0, 48), (0, 0), (0, 0)))
    dst = jnp.pad(edge_index[1].reshape(E // _CH, 1, _CH), ((0, 48), (0, 0), (0, 0)))

    h = _make_encoder(N, D, H)(x, W_enc, _rowvec(b_enc))

    seg = _make_sc_seg_sum(N, E, H)
    mlp = _make_gin_mlp(N, H)
    for l in range(L - 1):
        partials = seg(h, src, dst)
        h = mlp(partials, h, W1[l], _rowvec(b1[l]), _rowvec(g1[l]),
                _rowvec(be1[l]), W2[l], _rowvec(b2[l]))

    partials = seg(h, src, dst)
    wout_p = jnp.pad(Wout, ((0, 0), (0, H - C)))
    bout_p = _rowvec(jnp.pad(bout, (0, H - C)))
    ll = L - 1
    z = _make_gin_mlp_pool_head(N, H)(
        partials, h, W1[ll], _rowvec(b1[ll]), _rowvec(g1[ll]),
        _rowvec(be1[ll]), W2[ll], _rowvec(b2[ll]),
        batch.reshape(N, 1), Wf1, _rowvec(bf1), _rowvec(gf1),
        _rowvec(bef1), Wf2, _rowvec(bf2), _rowvec(gf2), _rowvec(bef2),
        wout_p, bout_p)
    return z[:, :C]


# speedup vs baseline: 12.0907x; 1.0164x over previous
import functools

import jax
import jax.numpy as jnp
from jax import lax
from jax.experimental import pallas as pl
from jax.experimental.pallas import tpu as pltpu
from jax.experimental.pallas import tpu_sc as plsc

_NC = 2
_NS = 16
_CH = 128
_BN = 1000
_G = 128


@functools.lru_cache(maxsize=None)
def _make_sc_seg_sum(N, E, H):
    n_chunks = E // _CH
    assert n_chunks * _CH == E
    ntiles = _NC * _NS
    nt_base = n_chunks // ntiles
    nt_rem = n_chunks - nt_base * ntiles
    nt_max = nt_base + (1 if nt_rem else 0)
    ph = 28
    nphases = -(-nt_max // ph)
    stage_rows = [min(ph, nt_max - p * ph) for p in range(nphases)]
    rps = (N // _NS) // 16 * 16
    tail = N - rps * _NS
    assert tail % 16 == 0 and tail <= 16

    mesh = plsc.VectorSubcoreMesh(
        core_axis_name="c", subcore_axis_name="s",
        num_cores=_NC, num_subcores=_NS)

    @functools.partial(
        pl.kernel,
        mesh=mesh,
        out_type=jax.ShapeDtypeStruct((_NC, N, H), jnp.float32),
        scratch_types=[
            pltpu.VMEM((2, ph, 1, _CH), jnp.int32),
            pltpu.VMEM((2, ph, 1, _CH), jnp.int32),
            pltpu.VMEM((2, _CH, H), jnp.float32),
            pltpu.VMEM((16, H), jnp.float32),
            pltpu.VMEM_SHARED((N, H), jnp.float32),
            pltpu.SemaphoreType.DMA,
            pltpu.SemaphoreType.DMA,
            pltpu.SemaphoreType.DMA,
        ],
    )
    def seg(h_hbm, edges_hbm, out_hbm, srcv, dstv, rowsv, zbuf, acc,
            gsem, isem, zsem):
        c = lax.axis_index("c")
        s = lax.axis_index("s")
        gid = c * _NS + s
        first_extra = ntiles - nt_rem
        nt = jnp.where(gid < first_extra, nt_base, nt_max)
        tbase = gid * nt_base + jnp.maximum(gid - first_extra, 0)
        zbase = s * rps

        def stage(p):
            sp = p % 2
            pbase = tbase + p * ph
            sr = stage_rows[p]
            return (pltpu.make_async_copy(
                        edges_hbm.at[0, pl.ds(pbase, sr)],
                        srcv.at[sp].at[pl.ds(0, sr)], isem),
                    pltpu.make_async_copy(
                        edges_hbm.at[1, pl.ds(pbase, sr)],
                        dstv.at[sp].at[pl.ds(0, sr)], isem))

        def gather(p, k, buf):
            return pltpu.make_async_copy(
                h_hbm.at[srcv.at[p % 2, k, 0]], rowsv.at[buf], gsem)

        def zero_copy(j):
            return pltpu.make_async_copy(
                zbuf, acc.at[pl.ds(zbase + j * 16, 16)], zsem)

        for d in stage(0):
            d.start()
        for r in range(16):
            for q in range(H // 16):
                zbuf[r, pl.ds(q * 16, 16)] = jnp.zeros((16,), jnp.float32)
        for d in stage(0):
            d.wait()
        gather(0, 0, 0).start()
        if nphases > 1:
            for d in stage(1):
                d.start()

        def zfire(j, carry):
            zero_copy(j).start()
            return carry

        lax.fori_loop(0, rps // 16, zfire, 0)

        def zdrain(j, carry):
            zero_copy(j).wait()
            return carry

        lax.fori_loop(0, rps // 16, zdrain, 0)
        if tail:
            @pl.when(s == _NS - 1)
            def _():
                pltpu.sync_copy(zbuf.at[pl.ds(0, tail)],
                                acc.at[pl.ds(N - tail, tail)])

        plsc.subcore_barrier()

        for p in range(nphases):
            cnt = jnp.minimum(nt - p * ph, ph)

            def edge_body(k, carry, p=p, cnt=cnt):
                @pl.when(k + 1 < cnt)
                def _():
                    gather(p, k + 1, (k + 1) % 2).start()
                gather(p, k, k % 2).wait()
                pltpu.sync_copy(rowsv.at[k % 2], acc.at[dstv.at[p % 2, k, 0]],
                                add=True)
                return carry

            lax.fori_loop(0, cnt, edge_body, 0)

            if p + 1 < nphases:
                for d in stage(p + 1):
                    d.wait()
                gather(p + 1, 0, 0).start()
                if p + 2 < nphases:
                    for d in stage(p + 2):
                        d.start()

        plsc.subcore_barrier()

        pltpu.sync_copy(acc.at[pl.ds(zbase, rps)],
                        out_hbm.at[c, pl.ds(zbase, rps)])
        if tail:
            @pl.when(s == _NS - 1)
            def _():
                pltpu.sync_copy(acc.at[pl.ds(N - tail, tail)],
                                out_hbm.at[c, pl.ds(N - tail, tail)])

    return seg


def _rowvec(v):
    return v.reshape(1, v.shape[0])


@functools.lru_cache(maxsize=None)
def _make_encoder(N, D, H):
    def body(x_ref, w_ref, b_ref, o_ref):
        o_ref[...] = (jnp.dot(x_ref[...], w_ref[...],
                              preferred_element_type=jnp.float32)
                      + b_ref[...])

    return pl.pallas_call(
        body,
        grid=(N // _BN,),
        in_specs=[
            pl.BlockSpec((_BN, D), lambda i: (i, 0)),
            pl.BlockSpec((D, H), lambda i: (0, 0)),
            pl.BlockSpec((1, H), lambda i: (0, 0)),
        ],
        out_specs=pl.BlockSpec((_BN, H), lambda i: (i, 0)),
        out_shape=jax.ShapeDtypeStruct((N, H), jnp.float32),
    )


@functools.lru_cache(maxsize=None)
def _make_gin_mlp(N, H):
    def body(p_ref, h_ref, w1, b1, g1, e1, w2, b2, o_ref):
        t = p_ref[0] + p_ref[1] + h_ref[...]
        t = jnp.dot(t, w1[...], preferred_element_type=jnp.float32) + b1[...]
        t = t * g1[...] + e1[...]
        t = jnp.maximum(t, 0.0)
        t = jnp.dot(t, w2[...], preferred_element_type=jnp.float32) + b2[...]
        o_ref[...] = jnp.maximum(t, 0.0)

    wspec = pl.BlockSpec((H, H), lambda i: (0, 0))
    bspec = pl.BlockSpec((1, H), lambda i: (0, 0))
    return pl.pallas_call(
        body,
        grid=(N // _BN,),
        in_specs=[
            pl.BlockSpec((_NC, _BN, H), lambda i: (0, i, 0)),
            pl.BlockSpec((_BN, H), lambda i: (i, 0)),
            wspec, bspec, bspec, bspec, wspec, bspec,
        ],
        out_specs=pl.BlockSpec((_BN, H), lambda i: (i, 0)),
        out_shape=jax.ShapeDtypeStruct((N, H), jnp.float32),
    )


@functools.lru_cache(maxsize=None)
def _make_gin_mlp_pool_head(N, H):
    nsteps = N // _BN

    def body(p_ref, h_ref, w1, b1, g1, e1, w2, b2, bt_ref,
             wf1, bf1, gf1, ef1, wf2, bf2, gf2, ef2, wo, bo, o_ref):
        i = pl.program_id(0)

        @pl.when(i == 0)
        def _():
            o_ref[...] = jnp.zeros_like(o_ref)

        t = p_ref[0] + p_ref[1] + h_ref[...]
        t = jnp.dot(t, w1[...], preferred_element_type=jnp.float32) + b1[...]
        t = t * g1[...] + e1[...]
        t = jnp.maximum(t, 0.0)
        t = jnp.dot(t, w2[...], preferred_element_type=jnp.float32) + b2[...]
        t = jnp.maximum(t, 0.0)

        onehot = (bt_ref[...] == lax.broadcasted_iota(
            jnp.int32, (_BN, _G), 1)).astype(jnp.float32)
        o_ref[...] += lax.dot_general(
            onehot, t, (((0,), (0,)), ((), ())),
            preferred_element_type=jnp.float32)

        @pl.when(i == nsteps - 1)
        def _():
            z = o_ref[...]
            z = jnp.dot(z, wf1[...], preferred_element_type=jnp.float32) + bf1[...]
            z = z * gf1[...] + ef1[...]
            z = jnp.maximum(z, 0.0)
            z = jnp.dot(z, wf2[...], preferred_element_type=jnp.float32) + bf2[...]
            z = z * gf2[...] + ef2[...]
            o_ref[...] = (jnp.dot(z, wo[...],
                                  preferred_element_type=jnp.float32)
                          + bo[...])

    wspec = pl.BlockSpec((H, H), lambda i: (0, 0))
    bspec = pl.BlockSpec((1, H), lambda i: (0, 0))
    return pl.pallas_call(
        body,
        grid=(nsteps,),
        in_specs=[
            pl.BlockSpec((_NC, _BN, H), lambda i: (0, i, 0)),
            pl.BlockSpec((_BN, H), lambda i: (i, 0)),
            wspec, bspec, bspec, bspec, wspec, bspec,
            pl.BlockSpec((_BN, 1), lambda i: (i, 0)),
            wspec, bspec, bspec, bspec, wspec, bspec, bspec, bspec,
            wspec, bspec,
        ],
        out_specs=pl.BlockSpec((_G, H), lambda i: (0, 0)),
        out_shape=jax.ShapeDtypeStruct((_G, H), jnp.float32),
    )


@functools.lru_cache(maxsize=None)
def _make_pool_head(N, H):
    nsteps = N // _BN

    def body(h_ref, b_ref, wf1, bf1, gf1, ef1, wf2, bf2, gf2, ef2, wo, bo,
             o_ref):
        i = pl.program_id(0)

        @pl.when(i == 0)
        def _():
            o_ref[...] = jnp.zeros_like(o_ref)

        onehot = (b_ref[...] == lax.broadcasted_iota(
            jnp.int32, (_BN, _G), 1)).astype(jnp.float32)
        o_ref[...] += lax.dot_general(
            onehot, h_ref[...], (((0,), (0,)), ((), ())),
            preferred_element_type=jnp.float32)

        @pl.when(i == nsteps - 1)
        def _():
            z = o_ref[...]
            z = jnp.dot(z, wf1[...], preferred_element_type=jnp.float32) + bf1[...]
            z = z * gf1[...] + ef1[...]
            z = jnp.maximum(z, 0.0)
            z = jnp.dot(z, wf2[...], preferred_element_type=jnp.float32) + bf2[...]
            z = z * gf2[...] + ef2[...]
            o_ref[...] = (jnp.dot(z, wo[...],
                                  preferred_element_type=jnp.float32)
                          + bo[...])

    wspec = pl.BlockSpec((H, H), lambda i: (0, 0))
    bspec = pl.BlockSpec((1, H), lambda i: (0, 0))
    return pl.pallas_call(
        body,
        grid=(nsteps,),
        in_specs=[
            pl.BlockSpec((_BN, H), lambda i: (i, 0)),
            pl.BlockSpec((_BN, 1), lambda i: (i, 0)),
            wspec, bspec, bspec, bspec, wspec, bspec, bspec, bspec,
            wspec, bspec,
        ],
        out_specs=pl.BlockSpec((_G, H), lambda i: (0, 0)),
        out_shape=jax.ShapeDtypeStruct((_G, H), jnp.float32),
    )


def kernel(x, edge_index, batch, W_enc, b_enc, W1, b1, g1, be1, W2, b2,
           Wf1, bf1, gf1, bef1, Wf2, bf2, gf2, bef2, Wout, bout):
    N, D = x.shape
    H = W_enc.shape[1]
    L = W1.shape[0]
    C = Wout.shape[1]
    E = edge_index.shape[1]

    edges = edge_index.reshape(2, E // _CH, 1, _CH)

    h = _make_encoder(N, D, H)(x, W_enc, _rowvec(b_enc))

    seg = _make_sc_seg_sum(N, E, H)
    mlp = _make_gin_mlp(N, H)
    for l in range(L - 1):
        partials = seg(h, edges)
        h = mlp(partials, h, W1[l], _rowvec(b1[l]), _rowvec(g1[l]),
                _rowvec(be1[l]), W2[l], _rowvec(b2[l]))

    partials = seg(h, edges)
    wout_p = jnp.pad(Wout, ((0, 0), (0, H - C)))
    bout_p = _rowvec(jnp.pad(bout, (0, H - C)))
    ll = L - 1
    z = _make_gin_mlp_pool_head(N, H)(
        partials, h, W1[ll], _rowvec(b1[ll]), _rowvec(g1[ll]),
        _rowvec(be1[ll]), W2[ll], _rowvec(b2[ll]),
        batch.reshape(N, 1), Wf1, _rowvec(bf1), _rowvec(gf1),
        _rowvec(bef1), Wf2, _rowvec(bf2), _rowvec(gf2), _rowvec(bef2),
        wout_p, bout_p)
    return z[:, :C]


# speedup vs baseline: 12.1028x; 1.0010x over previous
import functools

import jax
import jax.numpy as jnp
from jax import lax
from jax.experimental import pallas as pl
from jax.experimental.pallas import tpu as pltpu
from jax.experimental.pallas import tpu_sc as plsc

_NC = 2
_NS = 16
_CH = 128
_BN = 1000
_G = 128


@functools.lru_cache(maxsize=None)
def _make_sc_seg_sum(N, E, H):
    n_chunks = E // _CH
    assert n_chunks * _CH == E
    ntiles = _NC * _NS
    nt_base = n_chunks // ntiles
    nt_rem = n_chunks - nt_base * ntiles
    nt_max = nt_base + (1 if nt_rem else 0)
    ph = 28
    nphases = -(-nt_max // ph)
    stage_rows = [min(ph, nt_max - p * ph) for p in range(nphases)]
    rps = (N // _NS) // 16 * 16
    tail = N - rps * _NS
    assert tail % 16 == 0 and tail <= 16

    mesh = plsc.VectorSubcoreMesh(
        core_axis_name="c", subcore_axis_name="s",
        num_cores=_NC, num_subcores=_NS)

    @functools.partial(
        pl.kernel,
        mesh=mesh,
        out_type=jax.ShapeDtypeStruct((_NC, N, H), jnp.float32),
        scratch_types=[
            pltpu.VMEM((2, ph, 1, _CH), jnp.int32),
            pltpu.VMEM((2, ph, 1, _CH), jnp.int32),
            pltpu.VMEM((2, _CH, H), jnp.float32),
            pltpu.VMEM((16, H), jnp.float32),
            pltpu.VMEM_SHARED((N, H), jnp.float32),
            pltpu.SemaphoreType.DMA,
            pltpu.SemaphoreType.DMA,
            pltpu.SemaphoreType.DMA,
            pltpu.SemaphoreType.DMA,
        ],
    )
    def seg(h_hbm, edges_hbm, out_hbm, srcv, dstv, rowsv, zbuf, acc,
            gsem, isem, zsem, ssem):
        c = lax.axis_index("c")
        s = lax.axis_index("s")
        gid = c * _NS + s
        first_extra = ntiles - nt_rem
        nt = jnp.where(gid < first_extra, nt_base, nt_max)
        tbase = gid * nt_base + jnp.maximum(gid - first_extra, 0)
        zbase = s * rps

        def stage(p):
            sp = p % 2
            pbase = tbase + p * ph
            sr = stage_rows[p]
            return (pltpu.make_async_copy(
                        edges_hbm.at[0, pl.ds(pbase, sr)],
                        srcv.at[sp].at[pl.ds(0, sr)], isem),
                    pltpu.make_async_copy(
                        edges_hbm.at[1, pl.ds(pbase, sr)],
                        dstv.at[sp].at[pl.ds(0, sr)], isem))

        def gather(p, k, buf):
            return pltpu.make_async_copy(
                h_hbm.at[srcv.at[p % 2, k, 0]], rowsv.at[buf], gsem)

        def zero_copy(j):
            return pltpu.make_async_copy(
                zbuf, acc.at[pl.ds(zbase + j * 16, 16)], zsem)

        for d in stage(0):
            d.start()
        for r in range(16):
            for q in range(H // 16):
                zbuf[r, pl.ds(q * 16, 16)] = jnp.zeros((16,), jnp.float32)
        for d in stage(0):
            d.wait()
        gather(0, 0, 0).start()
        if nphases > 1:
            for d in stage(1):
                d.start()

        def zfire(j, carry):
            zero_copy(j).start()
            return carry

        lax.fori_loop(0, rps // 16, zfire, 0)

        def zdrain(j, carry):
            zero_copy(j).wait()
            return carry

        lax.fori_loop(0, rps // 16, zdrain, 0)
        if tail:
            @pl.when(s == _NS - 1)
            def _():
                pltpu.sync_copy(zbuf.at[pl.ds(0, tail)],
                                acc.at[pl.ds(N - tail, tail)])

        plsc.subcore_barrier()

        def scatter_start(p, k):
            pltpu.async_copy(rowsv.at[k % 2], acc.at[dstv.at[p % 2, k, 0]],
                             ssem, add=True)

        def scatter_wait(p, k):
            pltpu.make_async_copy(rowsv.at[k % 2],
                                  acc.at[dstv.at[p % 2, k, 0]], ssem).wait()

        for p in range(nphases):
            cnt = jnp.minimum(nt - p * ph, ph)

            def edge_body(k, carry, p=p, cnt=cnt):
                @pl.when(k >= 1)
                def _():
                    scatter_wait(p, k - 1)
                @pl.when(k + 1 < cnt)
                def _():
                    gather(p, k + 1, (k + 1) % 2).start()
                gather(p, k, k % 2).wait()
                scatter_start(p, k)
                return carry

            lax.fori_loop(0, cnt, edge_body, 0)
            scatter_wait(p, cnt - 1)

            if p + 1 < nphases:
                for d in stage(p + 1):
                    d.wait()
                gather(p + 1, 0, 0).start()
                if p + 2 < nphases:
                    for d in stage(p + 2):
                        d.start()

        plsc.subcore_barrier()

        pltpu.sync_copy(acc.at[pl.ds(zbase, rps)],
                        out_hbm.at[c, pl.ds(zbase, rps)])
        if tail:
            @pl.when(s == _NS - 1)
            def _():
                pltpu.sync_copy(acc.at[pl.ds(N - tail, tail)],
                                out_hbm.at[c, pl.ds(N - tail, tail)])

    return seg


def _rowvec(v):
    return v.reshape(1, v.shape[0])


@functools.lru_cache(maxsize=None)
def _make_encoder(N, D, H):
    def body(x_ref, w_ref, b_ref, o_ref):
        o_ref[...] = (jnp.dot(x_ref[...], w_ref[...],
                              preferred_element_type=jnp.float32)
                      + b_ref[...])

    return pl.pallas_call(
        body,
        grid=(N // _BN,),
        in_specs=[
            pl.BlockSpec((_BN, D), lambda i: (i, 0)),
            pl.BlockSpec((D, H), lambda i: (0, 0)),
            pl.BlockSpec((1, H), lambda i: (0, 0)),
        ],
        out_specs=pl.BlockSpec((_BN, H), lambda i: (i, 0)),
        out_shape=jax.ShapeDtypeStruct((N, H), jnp.float32),
    )


@functools.lru_cache(maxsize=None)
def _make_gin_mlp(N, H):
    def body(p_ref, h_ref, w1, b1, g1, e1, w2, b2, o_ref):
        t = p_ref[0] + p_ref[1] + h_ref[...]
        t = jnp.dot(t, w1[...], preferred_element_type=jnp.float32) + b1[...]
        t = t * g1[...] + e1[...]
        t = jnp.maximum(t, 0.0)
        t = jnp.dot(t, w2[...], preferred_element_type=jnp.float32) + b2[...]
        o_ref[...] = jnp.maximum(t, 0.0)

    wspec = pl.BlockSpec((H, H), lambda i: (0, 0))
    bspec = pl.BlockSpec((1, H), lambda i: (0, 0))
    return pl.pallas_call(
        body,
        grid=(N // _BN,),
        in_specs=[
            pl.BlockSpec((_NC, _BN, H), lambda i: (0, i, 0)),
            pl.BlockSpec((_BN, H), lambda i: (i, 0)),
            wspec, bspec, bspec, bspec, wspec, bspec,
        ],
        out_specs=pl.BlockSpec((_BN, H), lambda i: (i, 0)),
        out_shape=jax.ShapeDtypeStruct((N, H), jnp.float32),
    )


@functools.lru_cache(maxsize=None)
def _make_gin_mlp_pool_head(N, H):
    nsteps = N // _BN

    def body(p_ref, h_ref, w1, b1, g1, e1, w2, b2, bt_ref,
             wf1, bf1, gf1, ef1, wf2, bf2, gf2, ef2, wo, bo, o_ref):
        i = pl.program_id(0)

        @pl.when(i == 0)
        def _():
            o_ref[...] = jnp.zeros_like(o_ref)

        t = p_ref[0] + p_ref[1] + h_ref[...]
        t = jnp.dot(t, w1[...], preferred_element_type=jnp.float32) + b1[...]
        t = t * g1[...] + e1[...]
        t = jnp.maximum(t, 0.0)
        t = jnp.dot(t, w2[...], preferred_element_type=jnp.float32) + b2[...]
        t = jnp.maximum(t, 0.0)

        onehot = (bt_ref[...] == lax.broadcasted_iota(
            jnp.int32, (_BN, _G), 1)).astype(jnp.float32)
        o_ref[...] += lax.dot_general(
            onehot, t, (((0,), (0,)), ((), ())),
            preferred_element_type=jnp.float32)

        @pl.when(i == nsteps - 1)
        def _():
            z = o_ref[...]
            z = jnp.dot(z, wf1[...], preferred_element_type=jnp.float32) + bf1[...]
            z = z * gf1[...] + ef1[...]
            z = jnp.maximum(z, 0.0)
            z = jnp.dot(z, wf2[...], preferred_element_type=jnp.float32) + bf2[...]
            z = z * gf2[...] + ef2[...]
            o_ref[...] = (jnp.dot(z, wo[...],
                                  preferred_element_type=jnp.float32)
                          + bo[...])

    wspec = pl.BlockSpec((H, H), lambda i: (0, 0))
    bspec = pl.BlockSpec((1, H), lambda i: (0, 0))
    return pl.pallas_call(
        body,
        grid=(nsteps,),
        in_specs=[
            pl.BlockSpec((_NC, _BN, H), lambda i: (0, i, 0)),
            pl.BlockSpec((_BN, H), lambda i: (i, 0)),
            wspec, bspec, bspec, bspec, wspec, bspec,
            pl.BlockSpec((_BN, 1), lambda i: (i, 0)),
            wspec, bspec, bspec, bspec, wspec, bspec, bspec, bspec,
            wspec, bspec,
        ],
        out_specs=pl.BlockSpec((_G, H), lambda i: (0, 0)),
        out_shape=jax.ShapeDtypeStruct((_G, H), jnp.float32),
    )


@functools.lru_cache(maxsize=None)
def _make_pool_head(N, H):
    nsteps = N // _BN

    def body(h_ref, b_ref, wf1, bf1, gf1, ef1, wf2, bf2, gf2, ef2, wo, bo,
             o_ref):
        i = pl.program_id(0)

        @pl.when(i == 0)
        def _():
            o_ref[...] = jnp.zeros_like(o_ref)

        onehot = (b_ref[...] == lax.broadcasted_iota(
            jnp.int32, (_BN, _G), 1)).astype(jnp.float32)
        o_ref[...] += lax.dot_general(
            onehot, h_ref[...], (((0,), (0,)), ((), ())),
            preferred_element_type=jnp.float32)

        @pl.when(i == nsteps - 1)
        def _():
            z = o_ref[...]
            z = jnp.dot(z, wf1[...], preferred_element_type=jnp.float32) + bf1[...]
            z = z * gf1[...] + ef1[...]
            z = jnp.maximum(z, 0.0)
            z = jnp.dot(z, wf2[...], preferred_element_type=jnp.float32) + bf2[...]
            z = z * gf2[...] + ef2[...]
            o_ref[...] = (jnp.dot(z, wo[...],
                                  preferred_element_type=jnp.float32)
                          + bo[...])

    wspec = pl.BlockSpec((H, H), lambda i: (0, 0))
    bspec = pl.BlockSpec((1, H), lambda i: (0, 0))
    return pl.pallas_call(
        body,
        grid=(nsteps,),
        in_specs=[
            pl.BlockSpec((_BN, H), lambda i: (i, 0)),
            pl.BlockSpec((_BN, 1), lambda i: (i, 0)),
            wspec, bspec, bspec, bspec, wspec, bspec, bspec, bspec,
            wspec, bspec,
        ],
        out_specs=pl.BlockSpec((_G, H), lambda i: (0, 0)),
        out_shape=jax.ShapeDtypeStruct((_G, H), jnp.float32),
    )


def kernel(x, edge_index, batch, W_enc, b_enc, W1, b1, g1, be1, W2, b2,
           Wf1, bf1, gf1, bef1, Wf2, bf2, gf2, bef2, Wout, bout):
    N, D = x.shape
    H = W_enc.shape[1]
    L = W1.shape[0]
    C = Wout.shape[1]
    E = edge_index.shape[1]

    edges = edge_index.reshape(2, E // _CH, 1, _CH)

    h = _make_encoder(N, D, H)(x, W_enc, _rowvec(b_enc))

    seg = _make_sc_seg_sum(N, E, H)
    mlp = _make_gin_mlp(N, H)
    for l in range(L - 1):
        partials = seg(h, edges)
        h = mlp(partials, h, W1[l], _rowvec(b1[l]), _rowvec(g1[l]),
                _rowvec(be1[l]), W2[l], _rowvec(b2[l]))

    partials = seg(h, edges)
    wout_p = jnp.pad(Wout, ((0, 0), (0, H - C)))
    bout_p = _rowvec(jnp.pad(bout, (0, H - C)))
    ll = L - 1
    z = _make_gin_mlp_pool_head(N, H)(
        partials, h, W1[ll], _rowvec(b1[ll]), _rowvec(g1[ll]),
        _rowvec(be1[ll]), W2[ll], _rowvec(b2[ll]),
        batch.reshape(N, 1), Wf1, _rowvec(bf1), _rowvec(gf1),
        _rowvec(bef1), Wf2, _rowvec(bf2), _rowvec(gf2), _rowvec(bef2),
        wout_p, bout_p)
    return z[:, :C]


# speedup vs baseline: 12.4878x; 1.0318x over previous
import functools

import jax
import jax.numpy as jnp
from jax import lax
from jax.experimental import pallas as pl
from jax.experimental.pallas import tpu as pltpu
from jax.experimental.pallas import tpu_sc as plsc

_NC = 2
_NS = 16
_CH = 128
_BN = 2000
_G = 128


@functools.lru_cache(maxsize=None)
def _make_sc_seg_sum(N, E, H):
    n_chunks = E // _CH
    assert n_chunks * _CH == E
    ntiles = _NC * _NS
    nt_base = n_chunks // ntiles
    nt_rem = n_chunks - nt_base * ntiles
    nt_max = nt_base + (1 if nt_rem else 0)
    ph = 28
    nphases = -(-nt_max // ph)
    stage_rows = [min(ph, nt_max - p * ph) for p in range(nphases)]
    rps = (N // _NS) // 16 * 16
    tail = N - rps * _NS
    assert tail % 16 == 0 and tail <= 16

    mesh = plsc.VectorSubcoreMesh(
        core_axis_name="c", subcore_axis_name="s",
        num_cores=_NC, num_subcores=_NS)

    @functools.partial(
        pl.kernel,
        mesh=mesh,
        out_type=jax.ShapeDtypeStruct((_NC, N, H), jnp.float32),
        scratch_types=[
            pltpu.VMEM((2, ph, 1, _CH), jnp.int32),
            pltpu.VMEM((2, ph, 1, _CH), jnp.int32),
            pltpu.VMEM((2, _CH, H), jnp.float32),
            pltpu.VMEM((16, H), jnp.float32),
            pltpu.VMEM_SHARED((N, H), jnp.float32),
            pltpu.SemaphoreType.DMA,
            pltpu.SemaphoreType.DMA,
            pltpu.SemaphoreType.DMA,
            pltpu.SemaphoreType.DMA,
        ],
    )
    def seg(h_hbm, edges_hbm, out_hbm, srcv, dstv, rowsv, zbuf, acc,
            gsem, isem, zsem, ssem):
        c = lax.axis_index("c")
        s = lax.axis_index("s")
        gid = c * _NS + s
        first_extra = ntiles - nt_rem
        nt = jnp.where(gid < first_extra, nt_base, nt_max)
        tbase = gid * nt_base + jnp.maximum(gid - first_extra, 0)
        zbase = s * rps

        def stage(p):
            sp = p % 2
            pbase = tbase + p * ph
            sr = stage_rows[p]
            return (pltpu.make_async_copy(
                        edges_hbm.at[0, pl.ds(pbase, sr)],
                        srcv.at[sp].at[pl.ds(0, sr)], isem),
                    pltpu.make_async_copy(
                        edges_hbm.at[1, pl.ds(pbase, sr)],
                        dstv.at[sp].at[pl.ds(0, sr)], isem))

        def gather(p, k, buf):
            return pltpu.make_async_copy(
                h_hbm.at[srcv.at[p % 2, k, 0]], rowsv.at[buf], gsem)

        def zero_copy(j):
            return pltpu.make_async_copy(
                zbuf, acc.at[pl.ds(zbase + j * 16, 16)], zsem)

        for d in stage(0):
            d.start()
        for r in range(16):
            for q in range(H // 16):
                zbuf[r, pl.ds(q * 16, 16)] = jnp.zeros((16,), jnp.float32)
        for d in stage(0):
            d.wait()
        gather(0, 0, 0).start()
        if nphases > 1:
            for d in stage(1):
                d.start()

        def zfire(j, carry):
            zero_copy(j).start()
            return carry

        lax.fori_loop(0, rps // 16, zfire, 0)

        def zdrain(j, carry):
            zero_copy(j).wait()
            return carry

        lax.fori_loop(0, rps // 16, zdrain, 0)
        if tail:
            @pl.when(s == _NS - 1)
            def _():
                pltpu.sync_copy(zbuf.at[pl.ds(0, tail)],
                                acc.at[pl.ds(N - tail, tail)])

        plsc.subcore_barrier()

        def scatter_start(p, k):
            pltpu.async_copy(rowsv.at[k % 2], acc.at[dstv.at[p % 2, k, 0]],
                             ssem, add=True)

        def scatter_wait(p, k):
            pltpu.make_async_copy(rowsv.at[k % 2],
                                  acc.at[dstv.at[p % 2, k, 0]], ssem).wait()

        for p in range(nphases):
            cnt = jnp.minimum(nt - p * ph, ph)

            def edge_body(k, carry, p=p, cnt=cnt):
                @pl.when(k >= 1)
                def _():
                    scatter_wait(p, k - 1)
                @pl.when(k + 1 < cnt)
                def _():
                    gather(p, k + 1, (k + 1) % 2).start()
                gather(p, k, k % 2).wait()
                scatter_start(p, k)
                return carry

            lax.fori_loop(0, cnt, edge_body, 0)
            scatter_wait(p, cnt - 1)

            if p + 1 < nphases:
                for d in stage(p + 1):
                    d.wait()
                gather(p + 1, 0, 0).start()
                if p + 2 < nphases:
                    for d in stage(p + 2):
                        d.start()

        plsc.subcore_barrier()

        pltpu.sync_copy(acc.at[pl.ds(zbase, rps)],
                        out_hbm.at[c, pl.ds(zbase, rps)])
        if tail:
            @pl.when(s == _NS - 1)
            def _():
                pltpu.sync_copy(acc.at[pl.ds(N - tail, tail)],
                                out_hbm.at[c, pl.ds(N - tail, tail)])

    return seg


def _rowvec(v):
    return v.reshape(1, v.shape[0])


@functools.lru_cache(maxsize=None)
def _make_encoder(N, D, H):
    def body(x_ref, w_ref, b_ref, o_ref):
        o_ref[...] = (jnp.dot(x_ref[...], w_ref[...],
                              preferred_element_type=jnp.float32)
                      + b_ref[...])

    return pl.pallas_call(
        body,
        grid=(N // _BN,),
        in_specs=[
            pl.BlockSpec((_BN, D), lambda i: (i, 0)),
            pl.BlockSpec((D, H), lambda i: (0, 0)),
            pl.BlockSpec((1, H), lambda i: (0, 0)),
        ],
        out_specs=pl.BlockSpec((_BN, H), lambda i: (i, 0)),
        out_shape=jax.ShapeDtypeStruct((N, H), jnp.float32),
    )


@functools.lru_cache(maxsize=None)
def _make_gin_mlp(N, H):
    def body(p_ref, h_ref, w1, b1, g1, e1, w2, b2, o_ref):
        t = p_ref[0] + p_ref[1] + h_ref[...]
        t = jnp.dot(t, w1[...], preferred_element_type=jnp.float32) + b1[...]
        t = t * g1[...] + e1[...]
        t = jnp.maximum(t, 0.0)
        t = jnp.dot(t, w2[...], preferred_element_type=jnp.float32) + b2[...]
        o_ref[...] = jnp.maximum(t, 0.0)

    wspec = pl.BlockSpec((H, H), lambda i: (0, 0))
    bspec = pl.BlockSpec((1, H), lambda i: (0, 0))
    return pl.pallas_call(
        body,
        grid=(N // _BN,),
        in_specs=[
            pl.BlockSpec((_NC, _BN, H), lambda i: (0, i, 0)),
            pl.BlockSpec((_BN, H), lambda i: (i, 0)),
            wspec, bspec, bspec, bspec, wspec, bspec,
        ],
        out_specs=pl.BlockSpec((_BN, H), lambda i: (i, 0)),
        out_shape=jax.ShapeDtypeStruct((N, H), jnp.float32),
    )


@functools.lru_cache(maxsize=None)
def _make_gin_mlp_pool_head(N, H):
    nsteps = N // _BN

    def body(p_ref, h_ref, w1, b1, g1, e1, w2, b2, bt_ref,
             wf1, bf1, gf1, ef1, wf2, bf2, gf2, ef2, wo, bo, o_ref):
        i = pl.program_id(0)

        @pl.when(i == 0)
        def _():
            o_ref[...] = jnp.zeros_like(o_ref)

        t = p_ref[0] + p_ref[1] + h_ref[...]
        t = jnp.dot(t, w1[...], preferred_element_type=jnp.float32) + b1[...]
        t = t * g1[...] + e1[...]
        t = jnp.maximum(t, 0.0)
        t = jnp.dot(t, w2[...], preferred_element_type=jnp.float32) + b2[...]
        t = jnp.maximum(t, 0.0)

        onehot = (bt_ref[...] == lax.broadcasted_iota(
            jnp.int32, (_BN, _G), 1)).astype(jnp.float32)
        o_ref[...] += lax.dot_general(
            onehot, t, (((0,), (0,)), ((), ())),
            preferred_element_type=jnp.float32)

        @pl.when(i == nsteps - 1)
        def _():
            z = o_ref[...]
            z = jnp.dot(z, wf1[...], preferred_element_type=jnp.float32) + bf1[...]
            z = z * gf1[...] + ef1[...]
            z = jnp.maximum(z, 0.0)
            z = jnp.dot(z, wf2[...], preferred_element_type=jnp.float32) + bf2[...]
            z = z * gf2[...] + ef2[...]
            o_ref[...] = (jnp.dot(z, wo[...],
                                  preferred_element_type=jnp.float32)
                          + bo[...])

    wspec = pl.BlockSpec((H, H), lambda i: (0, 0))
    bspec = pl.BlockSpec((1, H), lambda i: (0, 0))
    return pl.pallas_call(
        body,
        grid=(nsteps,),
        in_specs=[
            pl.BlockSpec((_NC, _BN, H), lambda i: (0, i, 0)),
            pl.BlockSpec((_BN, H), lambda i: (i, 0)),
            wspec, bspec, bspec, bspec, wspec, bspec,
            pl.BlockSpec((_BN, 1), lambda i: (i, 0)),
            wspec, bspec, bspec, bspec, wspec, bspec, bspec, bspec,
            wspec, bspec,
        ],
        out_specs=pl.BlockSpec((_G, H), lambda i: (0, 0)),
        out_shape=jax.ShapeDtypeStruct((_G, H), jnp.float32),
    )


@functools.lru_cache(maxsize=None)
def _make_pool_head(N, H):
    nsteps = N // _BN

    def body(h_ref, b_ref, wf1, bf1, gf1, ef1, wf2, bf2, gf2, ef2, wo, bo,
             o_ref):
        i = pl.program_id(0)

        @pl.when(i == 0)
        def _():
            o_ref[...] = jnp.zeros_like(o_ref)

        onehot = (b_ref[...] == lax.broadcasted_iota(
            jnp.int32, (_BN, _G), 1)).astype(jnp.float32)
        o_ref[...] += lax.dot_general(
            onehot, h_ref[...], (((0,), (0,)), ((), ())),
            preferred_element_type=jnp.float32)

        @pl.when(i == nsteps - 1)
        def _():
            z = o_ref[...]
            z = jnp.dot(z, wf1[...], preferred_element_type=jnp.float32) + bf1[...]
            z = z * gf1[...] + ef1[...]
            z = jnp.maximum(z, 0.0)
            z = jnp.dot(z, wf2[...], preferred_element_type=jnp.float32) + bf2[...]
            z = z * gf2[...] + ef2[...]
            o_ref[...] = (jnp.dot(z, wo[...],
                                  preferred_element_type=jnp.float32)
                          + bo[...])

    wspec = pl.BlockSpec((H, H), lambda i: (0, 0))
    bspec = pl.BlockSpec((1, H), lambda i: (0, 0))
    return pl.pallas_call(
        body,
        grid=(nsteps,),
        in_specs=[
            pl.BlockSpec((_BN, H), lambda i: (i, 0)),
            pl.BlockSpec((_BN, 1), lambda i: (i, 0)),
            wspec, bspec, bspec, bspec, wspec, bspec, bspec, bspec,
            wspec, bspec,
        ],
        out_specs=pl.BlockSpec((_G, H), lambda i: (0, 0)),
        out_shape=jax.ShapeDtypeStruct((_G, H), jnp.float32),
    )


def kernel(x, edge_index, batch, W_enc, b_enc, W1, b1, g1, be1, W2, b2,
           Wf1, bf1, gf1, bef1, Wf2, bf2, gf2, bef2, Wout, bout):
    N, D = x.shape
    H = W_enc.shape[1]
    L = W1.shape[0]
    C = Wout.shape[1]
    E = edge_index.shape[1]

    edges = edge_index.reshape(2, E // _CH, 1, _CH)

    h = _make_encoder(N, D, H)(x, W_enc, _rowvec(b_enc))

    seg = _make_sc_seg_sum(N, E, H)
    mlp = _make_gin_mlp(N, H)
    for l in range(L - 1):
        partials = seg(h, edges)
        h = mlp(partials, h, W1[l], _rowvec(b1[l]), _rowvec(g1[l]),
                _rowvec(be1[l]), W2[l], _rowvec(b2[l]))

    partials = seg(h, edges)
    wout_p = jnp.pad(Wout, ((0, 0), (0, H - C)))
    bout_p = _rowvec(jnp.pad(bout, (0, H - C)))
    ll = L - 1
    z = _make_gin_mlp_pool_head(N, H)(
        partials, h, W1[ll], _rowvec(b1[ll]), _rowvec(g1[ll]),
        _rowvec(be1[ll]), W2[ll], _rowvec(b2[ll]),
        batch.reshape(N, 1), Wf1, _rowvec(bf1), _rowvec(gf1),
        _rowvec(bef1), Wf2, _rowvec(bf2), _rowvec(gf2), _rowvec(bef2),
        wout_p, bout_p)
    return z[:, :C]


# speedup vs baseline: 12.6599x; 1.0138x over previous
import functools

import jax
import jax.numpy as jnp
from jax import lax
from jax.experimental import pallas as pl
from jax.experimental.pallas import tpu as pltpu
from jax.experimental.pallas import tpu_sc as plsc

_NC = 2
_NS = 16
_CH = 128
_BN = 5000
_G = 128


@functools.lru_cache(maxsize=None)
def _make_sc_seg_sum(N, E, H):
    n_chunks = E // _CH
    assert n_chunks * _CH == E
    ntiles = _NC * _NS
    nt_base = n_chunks // ntiles
    nt_rem = n_chunks - nt_base * ntiles
    nt_max = nt_base + (1 if nt_rem else 0)
    ph = 28
    nphases = -(-nt_max // ph)
    stage_rows = [min(ph, nt_max - p * ph) for p in range(nphases)]
    rps = (N // _NS) // 16 * 16
    tail = N - rps * _NS
    assert tail % 16 == 0 and tail <= 16

    mesh = plsc.VectorSubcoreMesh(
        core_axis_name="c", subcore_axis_name="s",
        num_cores=_NC, num_subcores=_NS)

    @functools.partial(
        pl.kernel,
        mesh=mesh,
        out_type=jax.ShapeDtypeStruct((_NC, N, H), jnp.float32),
        scratch_types=[
            pltpu.VMEM((2, ph, 1, _CH), jnp.int32),
            pltpu.VMEM((2, ph, 1, _CH), jnp.int32),
            pltpu.VMEM((2, _CH, H), jnp.float32),
            pltpu.VMEM((16, H), jnp.float32),
            pltpu.VMEM_SHARED((N, H), jnp.float32),
            pltpu.SemaphoreType.DMA,
            pltpu.SemaphoreType.DMA,
            pltpu.SemaphoreType.DMA,
            pltpu.SemaphoreType.DMA,
        ],
    )
    def seg(h_hbm, edges_hbm, out_hbm, srcv, dstv, rowsv, zbuf, acc,
            gsem, isem, zsem, ssem):
        c = lax.axis_index("c")
        s = lax.axis_index("s")
        gid = c * _NS + s
        first_extra = ntiles - nt_rem
        nt = jnp.where(gid < first_extra, nt_base, nt_max)
        tbase = gid * nt_base + jnp.maximum(gid - first_extra, 0)
        zbase = s * rps

        def stage(p):
            sp = p % 2
            pbase = tbase + p * ph
            sr = stage_rows[p]
            return (pltpu.make_async_copy(
                        edges_hbm.at[0, pl.ds(pbase, sr)],
                        srcv.at[sp].at[pl.ds(0, sr)], isem),
                    pltpu.make_async_copy(
                        edges_hbm.at[1, pl.ds(pbase, sr)],
                        dstv.at[sp].at[pl.ds(0, sr)], isem))

        def gather(p, k, buf):
            return pltpu.make_async_copy(
                h_hbm.at[srcv.at[p % 2, k, 0]], rowsv.at[buf], gsem)

        def zero_copy(j):
            return pltpu.make_async_copy(
                zbuf, acc.at[pl.ds(zbase + j * 16, 16)], zsem)

        for d in stage(0):
            d.start()
        for r in range(16):
            for q in range(H // 16):
                zbuf[r, pl.ds(q * 16, 16)] = jnp.zeros((16,), jnp.float32)
        for d in stage(0):
            d.wait()
        gather(0, 0, 0).start()
        if nphases > 1:
            for d in stage(1):
                d.start()

        def zfire(j, carry):
            zero_copy(j).start()
            return carry

        lax.fori_loop(0, rps // 16, zfire, 0)

        def zdrain(j, carry):
            zero_copy(j).wait()
            return carry

        lax.fori_loop(0, rps // 16, zdrain, 0)
        if tail:
            @pl.when(s == _NS - 1)
            def _():
                pltpu.sync_copy(zbuf.at[pl.ds(0, tail)],
                                acc.at[pl.ds(N - tail, tail)])

        plsc.subcore_barrier()

        def scatter_start(p, k):
            pltpu.async_copy(rowsv.at[k % 2], acc.at[dstv.at[p % 2, k, 0]],
                             ssem, add=True)

        def scatter_wait(p, k):
            pltpu.make_async_copy(rowsv.at[k % 2],
                                  acc.at[dstv.at[p % 2, k, 0]], ssem).wait()

        for p in range(nphases):
            cnt = jnp.minimum(nt - p * ph, ph)

            def edge_body(k, carry, p=p, cnt=cnt):
                @pl.when(k >= 1)
                def _():
                    scatter_wait(p, k - 1)
                @pl.when(k + 1 < cnt)
                def _():
                    gather(p, k + 1, (k + 1) % 2).start()
                gather(p, k, k % 2).wait()
                scatter_start(p, k)
                return carry

            lax.fori_loop(0, cnt, edge_body, 0)
            scatter_wait(p, cnt - 1)

            if p + 1 < nphases:
                for d in stage(p + 1):
                    d.wait()
                gather(p + 1, 0, 0).start()
                if p + 2 < nphases:
                    for d in stage(p + 2):
                        d.start()

        plsc.subcore_barrier()

        pltpu.sync_copy(acc.at[pl.ds(zbase, rps)],
                        out_hbm.at[c, pl.ds(zbase, rps)])
        if tail:
            @pl.when(s == _NS - 1)
            def _():
                pltpu.sync_copy(acc.at[pl.ds(N - tail, tail)],
                                out_hbm.at[c, pl.ds(N - tail, tail)])

    return seg


def _rowvec(v):
    return v.reshape(1, v.shape[0])


@functools.lru_cache(maxsize=None)
def _make_encoder(N, D, H):
    def body(x_ref, w_ref, b_ref, o_ref):
        o_ref[...] = (jnp.dot(x_ref[...], w_ref[...],
                              preferred_element_type=jnp.float32)
                      + b_ref[...])

    return pl.pallas_call(
        body,
        grid=(N // _BN,),
        in_specs=[
            pl.BlockSpec((_BN, D), lambda i: (i, 0)),
            pl.BlockSpec((D, H), lambda i: (0, 0)),
            pl.BlockSpec((1, H), lambda i: (0, 0)),
        ],
        out_specs=pl.BlockSpec((_BN, H), lambda i: (i, 0)),
        out_shape=jax.ShapeDtypeStruct((N, H), jnp.float32),
    )


@functools.lru_cache(maxsize=None)
def _make_gin_mlp(N, H):
    def body(p_ref, h_ref, w1, b1, g1, e1, w2, b2, o_ref):
        t = p_ref[0] + p_ref[1] + h_ref[...]
        t = jnp.dot(t, w1[...], preferred_element_type=jnp.float32) + b1[...]
        t = t * g1[...] + e1[...]
        t = jnp.maximum(t, 0.0)
        t = jnp.dot(t, w2[...], preferred_element_type=jnp.float32) + b2[...]
        o_ref[...] = jnp.maximum(t, 0.0)

    wspec = pl.BlockSpec((H, H), lambda i: (0, 0))
    bspec = pl.BlockSpec((1, H), lambda i: (0, 0))
    return pl.pallas_call(
        body,
        grid=(N // _BN,),
        in_specs=[
            pl.BlockSpec((_NC, _BN, H), lambda i: (0, i, 0)),
            pl.BlockSpec((_BN, H), lambda i: (i, 0)),
            wspec, bspec, bspec, bspec, wspec, bspec,
        ],
        out_specs=pl.BlockSpec((_BN, H), lambda i: (i, 0)),
        out_shape=jax.ShapeDtypeStruct((N, H), jnp.float32),
    )


@functools.lru_cache(maxsize=None)
def _make_gin_mlp_pool_head(N, H):
    nsteps = N // _BN

    def body(p_ref, h_ref, w1, b1, g1, e1, w2, b2, bt_ref,
             wf1, bf1, gf1, ef1, wf2, bf2, gf2, ef2, wo, bo, o_ref):
        i = pl.program_id(0)

        @pl.when(i == 0)
        def _():
            o_ref[...] = jnp.zeros_like(o_ref)

        t = p_ref[0] + p_ref[1] + h_ref[...]
        t = jnp.dot(t, w1[...], preferred_element_type=jnp.float32) + b1[...]
        t = t * g1[...] + e1[...]
        t = jnp.maximum(t, 0.0)
        t = jnp.dot(t, w2[...], preferred_element_type=jnp.float32) + b2[...]
        t = jnp.maximum(t, 0.0)

        onehot = (bt_ref[...] == lax.broadcasted_iota(
            jnp.int32, (_BN, _G), 1)).astype(jnp.float32)
        o_ref[...] += lax.dot_general(
            onehot, t, (((0,), (0,)), ((), ())),
            preferred_element_type=jnp.float32)

        @pl.when(i == nsteps - 1)
        def _():
            z = o_ref[...]
            z = jnp.dot(z, wf1[...], preferred_element_type=jnp.float32) + bf1[...]
            z = z * gf1[...] + ef1[...]
            z = jnp.maximum(z, 0.0)
            z = jnp.dot(z, wf2[...], preferred_element_type=jnp.float32) + bf2[...]
            z = z * gf2[...] + ef2[...]
            o_ref[...] = (jnp.dot(z, wo[...],
                                  preferred_element_type=jnp.float32)
                          + bo[...])

    wspec = pl.BlockSpec((H, H), lambda i: (0, 0))
    bspec = pl.BlockSpec((1, H), lambda i: (0, 0))
    return pl.pallas_call(
        body,
        grid=(nsteps,),
        in_specs=[
            pl.BlockSpec((_NC, _BN, H), lambda i: (0, i, 0)),
            pl.BlockSpec((_BN, H), lambda i: (i, 0)),
            wspec, bspec, bspec, bspec, wspec, bspec,
            pl.BlockSpec((_BN, 1), lambda i: (i, 0)),
            wspec, bspec, bspec, bspec, wspec, bspec, bspec, bspec,
            wspec, bspec,
        ],
        out_specs=pl.BlockSpec((_G, H), lambda i: (0, 0)),
        out_shape=jax.ShapeDtypeStruct((_G, H), jnp.float32),
    )


@functools.lru_cache(maxsize=None)
def _make_pool_head(N, H):
    nsteps = N // _BN

    def body(h_ref, b_ref, wf1, bf1, gf1, ef1, wf2, bf2, gf2, ef2, wo, bo,
             o_ref):
        i = pl.program_id(0)

        @pl.when(i == 0)
        def _():
            o_ref[...] = jnp.zeros_like(o_ref)

        onehot = (b_ref[...] == lax.broadcasted_iota(
            jnp.int32, (_BN, _G), 1)).astype(jnp.float32)
        o_ref[...] += lax.dot_general(
            onehot, h_ref[...], (((0,), (0,)), ((), ())),
            preferred_element_type=jnp.float32)

        @pl.when(i == nsteps - 1)
        def _():
            z = o_ref[...]
            z = jnp.dot(z, wf1[...], preferred_element_type=jnp.float32) + bf1[...]
            z = z * gf1[...] + ef1[...]
            z = jnp.maximum(z, 0.0)
            z = jnp.dot(z, wf2[...], preferred_element_type=jnp.float32) + bf2[...]
            z = z * gf2[...] + ef2[...]
            o_ref[...] = (jnp.dot(z, wo[...],
                                  preferred_element_type=jnp.float32)
                          + bo[...])

    wspec = pl.BlockSpec((H, H), lambda i: (0, 0))
    bspec = pl.BlockSpec((1, H), lambda i: (0, 0))
    return pl.pallas_call(
        body,
        grid=(nsteps,),
        in_specs=[
            pl.BlockSpec((_BN, H), lambda i: (i, 0)),
            pl.BlockSpec((_BN, 1), lambda i: (i, 0)),
            wspec, bspec, bspec, bspec, wspec, bspec, bspec, bspec,
            wspec, bspec,
        ],
        out_specs=pl.BlockSpec((_G, H), lambda i: (0, 0)),
        out_shape=jax.ShapeDtypeStruct((_G, H), jnp.float32),
    )


def kernel(x, edge_index, batch, W_enc, b_enc, W1, b1, g1, be1, W2, b2,
           Wf1, bf1, gf1, bef1, Wf2, bf2, gf2, bef2, Wout, bout):
    N, D = x.shape
    H = W_enc.shape[1]
    L = W1.shape[0]
    C = Wout.shape[1]
    E = edge_index.shape[1]

    edges = edge_index.reshape(2, E // _CH, 1, _CH)

    h = _make_encoder(N, D, H)(x, W_enc, _rowvec(b_enc))

    seg = _make_sc_seg_sum(N, E, H)
    mlp = _make_gin_mlp(N, H)
    for l in range(L - 1):
        partials = seg(h, edges)
        h = mlp(partials, h, W1[l], _rowvec(b1[l]), _rowvec(g1[l]),
                _rowvec(be1[l]), W2[l], _rowvec(b2[l]))

    partials = seg(h, edges)
    wout_p = jnp.pad(Wout, ((0, 0), (0, H - C)))
    bout_p = _rowvec(jnp.pad(bout, (0, H - C)))
    ll = L - 1
    z = _make_gin_mlp_pool_head(N, H)(
        partials, h, W1[ll], _rowvec(b1[ll]), _rowvec(g1[ll]),
        _rowvec(be1[ll]), W2[ll], _rowvec(b2[ll]),
        batch.reshape(N, 1), Wf1, _rowvec(bf1), _rowvec(gf1),
        _rowvec(bef1), Wf2, _rowvec(bf2), _rowvec(gf2), _rowvec(bef2),
        wout_p, bout_p)
    return z[:, :C]
